# baseline (speedup 1.0000x reference)
import functools
import math

import jax
import jax.numpy as jnp
from jax import lax
from jax.experimental import pallas as pl
from jax.experimental.pallas import tpu as pltpu
from jax.experimental.pallas import tpu_sc as plsc

API_DIM = 32
CLASS_DIM = 32
FINAL_DIM = API_DIM + CLASS_DIM
SCALE = math.sqrt(FINAL_DIM)

NC = 2
NS = 16
NW = NC * NS
CHUNK = 128


def _sc_embed(batch, hist):
    n_per_w = batch // NW
    k_per_h = n_per_w // CHUNK
    n_chunks = hist * k_per_h
    mesh = plsc.VectorSubcoreMesh(core_axis_name="c", subcore_axis_name="s")

    @functools.partial(
        pl.kernel,
        out_type=jax.ShapeDtypeStruct((hist, FINAL_DIM, batch), jnp.float32),
        mesh=mesh,
        scratch_types=[
            pltpu.VMEM((hist, n_per_w), jnp.int32),
            pltpu.VMEM((hist, n_per_w), jnp.int32),
            pltpu.VMEM((CHUNK, CLASS_DIM), jnp.float32),
            pltpu.VMEM((CHUNK, API_DIM), jnp.float32),
            pltpu.VMEM((FINAL_DIM, CHUNK), jnp.float32),
            pltpu.VMEM((CHUNK, CLASS_DIM), jnp.float32),
            pltpu.VMEM((CHUNK, API_DIM), jnp.float32),
            pltpu.VMEM((FINAL_DIM, CHUNK), jnp.float32),
            pltpu.SemaphoreType.DMA,
            pltpu.SemaphoreType.DMA,
            pltpu.SemaphoreType.DMA,
            pltpu.SemaphoreType.DMA,
        ],
        compiler_params=pltpu.CompilerParams(
            use_tc_tiling_on_sc=False, needs_layout_passes=False),
    )
    def k(cls_ids, api_ids, cls_tab, api_tab, out,
          idx_cls, idx_api, cls_v0, api_v0, out_v0, cls_v1, api_v1, out_v1,
          sem_g0, sem_g1, sem_w0, sem_w1):
        wid = lax.axis_index("s") * NC + lax.axis_index("c")
        nbase = wid * n_per_w
        pltpu.sync_copy(cls_ids.at[:, pl.ds(nbase, n_per_w)], idx_cls)
        pltpu.sync_copy(api_ids.at[:, pl.ds(nbase, n_per_w)], idx_api)

        bufs = ((cls_v0, api_v0, out_v0, sem_g0, sem_w0),
                (cls_v1, api_v1, out_v1, sem_g1, sem_w1))

        def gather_start(j, p):
            h, kk = j // k_per_h, j % k_per_h
            cls_v, api_v, _, sem_g, _ = bufs[p]
            pltpu.async_copy(
                cls_tab.at[idx_cls.at[h, pl.ds(kk * CHUNK, CHUNK)]],
                cls_v, sem_g)
            pltpu.async_copy(
                api_tab.at[idx_api.at[h, pl.ds(kk * CHUNK, CHUNK)]],
                api_v, sem_g)

        gather_start(0, 0)
        iota16 = jax.lax.broadcasted_iota(jnp.int32, (16,), 0)

        def pair_body(jj, carry):
            for p in range(2):
                j = jj * 2 + p
                h, kk = j // k_per_h, j % k_per_h
                cls_v, api_v, out_v, sem_g, sem_w = bufs[p]

                @pl.when(j + 1 < n_chunks)
                def _():
                    gather_start(j + 1, 1 - p)

                pltpu.make_async_copy(
                    cls_tab.at[idx_cls.at[0, pl.ds(0, CHUNK)]],
                    cls_v, sem_g).wait()
                pltpu.make_async_copy(
                    api_tab.at[idx_api.at[0, pl.ds(0, CHUNK)]],
                    api_v, sem_g).wait()

                @pl.when(j >= 2)
                def _():
                    pltpu.make_async_copy(
                        out_v, out.at[0, :, pl.ds(nbase, CHUNK)], sem_w).wait()

                def jg_body(jg, c):
                    rowv = iota16 + jg * 16
                    for d in range(CLASS_DIM):
                        dv = jnp.full((16,), d, jnp.int32)
                        out_v[d, pl.ds(jg * 16, 16)] = (
                            plsc.load_gather(cls_v, [rowv, dv]) * SCALE)
                    for d in range(API_DIM):
                        dv = jnp.full((16,), d, jnp.int32)
                        out_v[CLASS_DIM + d, pl.ds(jg * 16, 16)] = (
                            plsc.load_gather(api_v, [rowv, dv]) * SCALE)
                    return c

                lax.fori_loop(0, CHUNK // 16, jg_body, 0)

                pltpu.async_copy(
                    out_v,
                    out.at[h, :, pl.ds(nbase + kk * CHUNK, CHUNK)], sem_w)
            return carry

        lax.fori_loop(0, n_chunks // 2, pair_body, 0)
        for p in range(2):
            _, _, out_v, _, sem_w = bufs[p]
            pltpu.make_async_copy(
                out_v, out.at[0, :, pl.ds(nbase, CHUNK)], sem_w).wait()

    return k


def kernel(class_ids, api_ids, class_table, api_table):
    batch, hist = class_ids.shape
    assert batch % (NW * CHUNK) == 0
    cls_t = class_ids.T.astype(jnp.int32)
    api_t = api_ids.T.astype(jnp.int32)
    out = _sc_embed(batch, hist)(cls_t, api_t, class_table, api_table)
    return out.transpose(2, 0, 1)


# speedup vs baseline: 1.0656x; 1.0656x over previous
import functools
import math

import jax
import jax.numpy as jnp
from jax import lax
from jax.experimental import pallas as pl
from jax.experimental.pallas import tpu as pltpu
from jax.experimental.pallas import tpu_sc as plsc

API_DIM = 32
CLASS_DIM = 32
FINAL_DIM = API_DIM + CLASS_DIM
SCALE = math.sqrt(FINAL_DIM)

NC = 2
NS = 16
NW = NC * NS
CHUNK = 128
GROUP = 128 // API_DIM


def _sc_embed(batch, hist, cls_vocab4, api_vocab4):
    n_per_w = batch // NW
    k_per_h = n_per_w // CHUNK
    n_chunks = hist * k_per_h
    mesh = plsc.VectorSubcoreMesh(core_axis_name="c", subcore_axis_name="s")

    @functools.partial(
        pl.kernel,
        out_type=jax.ShapeDtypeStruct((hist, FINAL_DIM, batch), jnp.float32),
        mesh=mesh,
        scratch_types=[
            pltpu.VMEM((n_chunks, CHUNK), jnp.int32),
            pltpu.VMEM((n_chunks, CHUNK), jnp.int32),
            pltpu.VMEM((CHUNK, 128), jnp.float32),
            pltpu.VMEM((CHUNK, 128), jnp.float32),
            pltpu.VMEM((FINAL_DIM, CHUNK), jnp.float32),
            pltpu.VMEM((CHUNK, 128), jnp.float32),
            pltpu.VMEM((CHUNK, 128), jnp.float32),
            pltpu.VMEM((FINAL_DIM, CHUNK), jnp.float32),
            pltpu.VMEM((CHUNK,), jnp.int32),
            pltpu.VMEM((CHUNK,), jnp.int32),
            pltpu.VMEM((CHUNK,), jnp.int32),
            pltpu.VMEM((CHUNK,), jnp.int32),
            pltpu.SemaphoreType.DMA,
            pltpu.SemaphoreType.DMA,
            pltpu.SemaphoreType.DMA,
            pltpu.SemaphoreType.DMA,
        ],
        compiler_params=pltpu.CompilerParams(
            use_tc_tiling_on_sc=True, needs_layout_passes=False),
    )
    def k(cls_idx, api_idx, cls_tab, api_tab, out,
          idxc_c, idxc_a, g_c0, g_a0, out_v0, g_c1, g_a1, out_v1,
          idxg_c0, idxg_a0, idxg_c1, idxg_a1,
          sem_g0, sem_g1, sem_w0, sem_w1):
        wid = lax.axis_index("s") * NC + lax.axis_index("c")
        nbase = wid * n_per_w
        pltpu.sync_copy(cls_idx.at[wid], idxc_c)
        pltpu.sync_copy(api_idx.at[wid], idxc_a)

        bufs = ((g_c0, g_a0, out_v0, idxg_c0, idxg_a0, sem_g0, sem_w0),
                (g_c1, g_a1, out_v1, idxg_c1, idxg_a1, sem_g1, sem_w1))

        def gather_start(j, p):
            g_c, g_a, _, idxg_c, idxg_a, sem_g, _ = bufs[p]
            for v8 in range(CHUNK // 16):
                sl = pl.ds(v8 * 16, 16)
                idxg_c[sl] = lax.shift_right_logical(idxc_c[j, sl], 2)
                idxg_a[sl] = lax.shift_right_logical(idxc_a[j, sl], 2)
            pltpu.async_copy(cls_tab.at[idxg_c], g_c, sem_g)
            pltpu.async_copy(api_tab.at[idxg_a], g_a, sem_g)

        gather_start(0, 0)
        iota16 = jax.lax.broadcasted_iota(jnp.int32, (16,), 0)

        def pair_body(jj, carry):
            for p in range(2):
                j = jj * 2 + p
                h, kk = j // k_per_h, j % k_per_h
                g_c, g_a, out_v, idxg_c, idxg_a, sem_g, sem_w = bufs[p]

                @pl.when(j + 1 < n_chunks)
                def _():
                    gather_start(j + 1, 1 - p)

                pltpu.make_async_copy(cls_tab.at[idxg_c], g_c, sem_g).wait()
                pltpu.make_async_copy(api_tab.at[idxg_a], g_a, sem_g).wait()

                @pl.when(j >= 2)
                def _():
                    pltpu.make_async_copy(
                        out_v, out.at[0, :, pl.ds(nbase, CHUNK)], sem_w).wait()

                def jg_body(jg, c):
                    sl = pl.ds(jg * 16, 16)
                    rowv = iota16 + jg * 16
                    sub_c = lax.shift_left(
                        lax.bitwise_and(idxc_c[j, sl], 3), 5)
                    sub_a = lax.shift_left(
                        lax.bitwise_and(idxc_a[j, sl], 3), 5)
                    for d in range(CLASS_DIM):
                        out_v[d, sl] = (
                            plsc.load_gather(g_c, [rowv, sub_c + d]) * SCALE)
                    for d in range(API_DIM):
                        out_v[CLASS_DIM + d, sl] = (
                            plsc.load_gather(g_a, [rowv, sub_a + d]) * SCALE)
                    return c

                lax.fori_loop(0, CHUNK // 16, jg_body, 0)

                pltpu.async_copy(
                    out_v,
                    out.at[h, :, pl.ds(nbase + kk * CHUNK, CHUNK)], sem_w)
            return carry

        lax.fori_loop(0, n_chunks // 2, pair_body, 0)
        for p in range(2):
            out_v, sem_w = bufs[p][2], bufs[p][6]
            pltpu.make_async_copy(
                out_v, out.at[0, :, pl.ds(nbase, CHUNK)], sem_w).wait()

    return k


def kernel(class_ids, api_ids, class_table, api_table):
    batch, hist = class_ids.shape
    assert batch % (NW * CHUNK) == 0
    k_per_h = batch // (NW * CHUNK)

    def prep_ids(ids):
        t = ids.T.astype(jnp.int32)
        return (t.reshape(hist, NW, k_per_h, CHUNK)
                 .transpose(1, 0, 2, 3)
                 .reshape(NW, hist * k_per_h, CHUNK))

    cls_vocab, _ = class_table.shape
    api_vocab, _ = api_table.shape
    rm_cls = class_table.reshape(cls_vocab // GROUP, 128)
    rm_api = api_table.reshape(api_vocab // GROUP, 128)
    out = _sc_embed(batch, hist, cls_vocab // GROUP, api_vocab // GROUP)(
        prep_ids(class_ids), prep_ids(api_ids), rm_cls, rm_api)
    return out.transpose(2, 0, 1)
